# baseline (speedup 1.0000x reference)
import functools

import jax
import jax.numpy as jnp
from jax import lax
from jax.experimental import pallas as pl
from jax.experimental.pallas import tpu as pltpu
from jax.experimental.pallas import tpu_sc as plsc

N_NODES = 10000
D = 128
NP = 10240
E = 320000
B = 4096
S = 20
NC, NS = 2, 16
NW = NC * NS
CHUNK = 128
EPW_CH = 80
EP = NW * EPW_CH * CHUNK
IDX_BLK = 8
G_TOT = B * (S + 1)
GPW_CH = 24
G_PAD = NW * GPW_CH * CHUNK

_mesh = plsc.VectorSubcoreMesh(core_axis_name="c", subcore_axis_name="s")


@functools.partial(
    pl.kernel,
    out_type=jax.ShapeDtypeStruct((NC * NP,), jnp.float32),
    mesh=_mesh,
    scratch_types=[
        pltpu.VMEM((IDX_BLK, CHUNK), jnp.int32),
        pltpu.VMEM((CHUNK,), jnp.float32),
        pltpu.VMEM_SHARED((NP,), jnp.float32),
    ],
)
def _deg_kernel(dst2d_hbm, zvec_hbm, deg_hbm, idx_v, ones_v, deg_sh):
    cid = lax.axis_index("c")
    sid = lax.axis_index("s")
    wid = sid * NC + cid
    for j in range(CHUNK // 16):
        ones_v[pl.ds(j * 16, 16)] = jnp.full((16,), 1.0, jnp.float32)
    rows = NP // NS
    pltpu.sync_copy(zvec_hbm.at[pl.ds(sid * rows, rows)],
                    deg_sh.at[pl.ds(sid * rows, rows)])
    plsc.subcore_barrier()

    rbase = wid * EPW_CH

    def outer(ob, carry):
        row0 = rbase + ob * IDX_BLK
        pltpu.sync_copy(dst2d_hbm.at[pl.ds(row0, IDX_BLK)], idx_v)
        for j in range(IDX_BLK):
            pltpu.sync_copy(ones_v, deg_sh.at[idx_v.at[j]], add=True)
        return carry

    lax.fori_loop(0, EPW_CH // IDX_BLK, outer, 0)
    plsc.subcore_barrier()
    pltpu.sync_copy(deg_sh.at[pl.ds(sid * rows, rows)],
                    deg_hbm.at[pl.ds(cid * NP + sid * rows, rows)])


NBUF = 4
IDXB = 16


@functools.partial(
    pl.kernel,
    out_type=jax.ShapeDtypeStruct((NC * NP, D), jnp.float32),
    mesh=_mesh,
    scratch_types=[
        pltpu.VMEM((IDXB, CHUNK), jnp.int32),
        pltpu.VMEM((IDXB, CHUNK), jnp.int32),
        pltpu.VMEM((CHUNK, D), jnp.float32),
        pltpu.VMEM((CHUNK, D), jnp.float32),
        pltpu.VMEM_SHARED((NP, D), jnp.float32),
        pltpu.SemaphoreType.DMA,
        pltpu.SemaphoreType.DMA,
        pltpu.SemaphoreType.DMA,
        pltpu.SemaphoreType.DMA,
    ],
)
def _scatter_kernel(g_hbm, src2d_hbm, dst2d_hbm, zrows_hbm, acc_hbm,
                    sidx, didx, r0, r1, acc_sh, gs0, gs1, ss0, ss1):
    cid = lax.axis_index("c")
    sid = lax.axis_index("s")
    wid = sid * NC + cid
    rbase = wid * EPW_CH
    rows_b = (r0, r1)
    gsem = (gs0, gs1)
    ssem = (ss0, ss1)

    for r in range(NP // CHUNK // NS):
        row0 = (sid * (NP // CHUNK // NS) + r) * CHUNK
        pltpu.sync_copy(zrows_hbm, acc_sh.at[pl.ds(row0, CHUNK)])
    plsc.subcore_barrier()

    def g_start(k, b):
        pltpu.async_copy(g_hbm.at[sidx.at[k]], rows_b[b], gsem[b])

    def g_wait(b):
        pltpu.make_async_copy(g_hbm.at[sidx.at[0]], rows_b[b], gsem[b]).wait()

    def s_start(k, b):
        pltpu.sync_copy(rows_b[b], acc_sh.at[didx.at[k]], add=True)

    def s_wait(b):
        pass

    def block(ob, carry):
        row0 = rbase + ob * IDXB
        pltpu.sync_copy(src2d_hbm.at[pl.ds(row0, IDXB)], sidx)
        pltpu.sync_copy(dst2d_hbm.at[pl.ds(row0, IDXB)], didx)
        g_start(0, 0)
        g_wait(0)
        s_start(0, 0)
        g_start(1, 1)

        def inner(p, c):
            k1 = 2 * p + 1
            g_wait(1)
            s_wait(0)
            s_start(k1, 1)
            g_start(k1 + 1, 0)
            g_wait(0)
            s_wait(1)
            s_start(k1 + 1, 0)
            g_start(k1 + 2, 1)
            return c

        lax.fori_loop(0, (IDXB - 2) // 2, inner, 0)
        g_wait(1)
        s_wait(0)
        s_start(IDXB - 1, 1)
        s_wait(1)
        return carry

    lax.fori_loop(0, EPW_CH // IDXB, block, 0)
    plsc.subcore_barrier()
    for r in range(NP // CHUNK // NS):
        row0 = (sid * (NP // CHUNK // NS) + r) * CHUNK
        pltpu.sync_copy(acc_sh.at[pl.ds(row0, CHUNK)],
                        acc_hbm.at[pl.ds(cid * NP + row0, CHUNK)])


IPSB = 4
SPSB = IPSB * S
NSB = (B // NW) // IPSB


@functools.partial(
    pl.kernel,
    out_type=jax.ShapeDtypeStruct((B * S,), jnp.float32),
    mesh=_mesh,
    scratch_types=[
        pltpu.VMEM((B // NW,), jnp.int32),
        pltpu.VMEM((B // NW * S,), jnp.int32),
        pltpu.VMEM((B // NW, D), jnp.float32),
        pltpu.VMEM((SPSB, D), jnp.float32),
        pltpu.VMEM((SPSB, D), jnp.float32),
        pltpu.VMEM((16, 16), jnp.float32),
        pltpu.VMEM((B // NW * S,), jnp.float32),
        pltpu.SemaphoreType.DMA,
        pltpu.SemaphoreType.DMA,
        pltpu.SemaphoreType.DMA,
    ],
)
def _score_kernel(upd_hbm, items_hbm, samps_hbm, out_hbm,
                  it_idx, sp_idx, itbuf, smp0, smp1, stage, score,
                  isem, gs0, gs1):
    cid = lax.axis_index("c")
    sid = lax.axis_index("s")
    wid = sid * NC + cid
    ipw = B // NW
    spw = ipw * S
    smp_b = (smp0, smp1)
    gsem = (gs0, gs1)

    pltpu.sync_copy(items_hbm.at[pl.ds(wid * ipw, ipw)], it_idx)
    pltpu.sync_copy(samps_hbm.at[pl.ds(wid * spw, spw)], sp_idx)
    pltpu.async_copy(upd_hbm.at[it_idx], itbuf, isem).wait()

    rows16 = lax.iota(jnp.int32, 16)
    perms = [rows16 ^ k for k in (8, 4, 2, 1)]

    def g_start(sb, b):
        pltpu.async_copy(upd_hbm.at[sp_idx.at[pl.ds(sb * SPSB, SPSB)]],
                         smp_b[b], gsem[b])

    def g_wait(b):
        pltpu.make_async_copy(upd_hbm.at[sp_idx.at[pl.ds(0, SPSB)]],
                              smp_b[b], gsem[b]).wait()

    def compute(sb, b):
        buf = smp_b[b]
        vacc = jnp.zeros((16,), jnp.float32)
        for i in range(IPSB):
            irow = sb * IPSB + i
            itv = [itbuf[irow, pl.ds(v * 16, 16)] for v in range(8)]
            for s in range(S):
                d = i * S + s
                t = itv[0] * buf[d, pl.ds(0, 16)]
                for v in range(1, 8):
                    t = t + itv[v] * buf[d, pl.ds(v * 16, 16)]
                for p in perms:
                    t = t + t[p]
                vacc = jnp.where(rows16 == (d % 16), t, vacc)
                if d % 16 == 15:
                    score[pl.ds(sb * SPSB + (d // 16) * 16, 16)] = vacc

    g_start(0, 0)

    def body(g, carry):
        sb = 2 * g
        g_start(sb + 1, 1)
        g_wait(0)
        compute(sb, 0)
        g_start(sb + 2, 0)
        g_wait(1)
        compute(sb + 1, 1)
        return carry

    lax.fori_loop(0, NSB // 2 - 1, body, 0)
    g_start(NSB - 1, 1)
    g_wait(0)
    compute(NSB - 2, 0)
    g_wait(1)
    compute(NSB - 1, 1)

    pltpu.sync_copy(score, out_hbm.at[pl.ds(wid * spw, spw)])


_RB = 2048


def _tc1_body(emb_ref, w_ref, deg_ref, g_ref, dinv_ref):
    deg = deg_ref[0] + deg_ref[1] + 1.0
    dinv = lax.rsqrt(deg)
    dinv_ref[...] = dinv
    h = jnp.dot(emb_ref[...], w_ref[...], preferred_element_type=jnp.float32)
    g_ref[...] = h * dinv


_tc1 = pl.pallas_call(
    _tc1_body,
    grid=(NP // _RB,),
    in_specs=[
        pl.BlockSpec((_RB, D), lambda i: (i, 0)),
        pl.BlockSpec((D, D), lambda i: (0, 0)),
        pl.BlockSpec((NC, _RB, 1), lambda i: (0, i, 0)),
    ],
    out_specs=[
        pl.BlockSpec((_RB, D), lambda i: (i, 0)),
        pl.BlockSpec((_RB, 1), lambda i: (i, 0)),
    ],
    out_shape=[
        jax.ShapeDtypeStruct((NP, D), jnp.float32),
        jax.ShapeDtypeStruct((NP, 1), jnp.float32),
    ],
)


def _tc2_body(acc_ref, g_ref, dinv_ref, b_ref, upd_ref):
    upd_ref[...] = (dinv_ref[...] * (acc_ref[0] + acc_ref[1] + g_ref[...])
                    + b_ref[...])


_tc2 = pl.pallas_call(
    _tc2_body,
    grid=(NP // _RB,),
    in_specs=[
        pl.BlockSpec((NC, _RB, D), lambda i: (0, i, 0)),
        pl.BlockSpec((_RB, D), lambda i: (i, 0)),
        pl.BlockSpec((_RB, 1), lambda i: (i, 0)),
        pl.BlockSpec((1, D), lambda i: (0, 0)),
    ],
    out_specs=pl.BlockSpec((_RB, D), lambda i: (i, 0)),
    out_shape=jax.ShapeDtypeStruct((NP, D), jnp.float32),
)

def kernel(items, samples, edge_index, emb_weight, W, b):
    f32 = jnp.float32
    i32 = jnp.int32
    src = edge_index[0].astype(i32)
    dst = edge_index[1].astype(i32)
    npad = EP - E
    pad = (jnp.arange(npad, dtype=i32) % (NP - N_NODES)) + N_NODES
    src2d = jnp.concatenate([src, pad]).reshape(EP // CHUNK, CHUNK)
    dst2d = jnp.concatenate([dst, pad]).reshape(EP // CHUNK, CHUNK)
    emb_p = jnp.pad(emb_weight.astype(f32), ((0, NP - N_NODES), (0, 0)))

    zvec = jnp.zeros((NP,), f32)
    zrows = jnp.zeros((CHUNK, D), f32)

    deg = _deg_kernel(dst2d, zvec)
    g, dinv = _tc1(emb_p, W.astype(f32), deg.reshape(NC, NP, 1))
    acc = _scatter_kernel(g, src2d, dst2d, zrows)
    upd = _tc2(acc.reshape(NC, NP, D), g, dinv, b.astype(f32).reshape(1, D))

    flat = _score_kernel(upd, items.astype(i32),
                         samples.astype(i32).reshape(-1))
    return flat.reshape(B, S)


# speedup vs baseline: 1.1704x; 1.1704x over previous
import functools

import jax
import jax.numpy as jnp
from jax import lax
from jax.experimental import pallas as pl
from jax.experimental.pallas import tpu as pltpu
from jax.experimental.pallas import tpu_sc as plsc

N_NODES = 10000
D = 128
NP = 10240
E = 320000
B = 4096
S = 20
NC, NS = 2, 16
NW = NC * NS
CHUNK = 128
EC = 80
ECPW = 128
EP = NW * ECPW * EC
IDX_BLK = 8
G_TOT = B * (S + 1)
GPW_CH = 24
G_PAD = NW * GPW_CH * CHUNK

_mesh = plsc.VectorSubcoreMesh(core_axis_name="c", subcore_axis_name="s")


@functools.partial(
    pl.kernel,
    out_type=jax.ShapeDtypeStruct((NC * NP,), jnp.float32),
    mesh=_mesh,
    scratch_types=[
        pltpu.VMEM((IDX_BLK, EC), jnp.int32),
        pltpu.VMEM((EC,), jnp.float32),
        pltpu.VMEM_SHARED((NP,), jnp.float32),
    ],
)
def _deg_kernel(dst2d_hbm, zvec_hbm, deg_hbm, idx_v, ones_v, deg_sh):
    cid = lax.axis_index("c")
    sid = lax.axis_index("s")
    wid = sid * NC + cid
    for j in range(EC // 16):
        ones_v[pl.ds(j * 16, 16)] = jnp.full((16,), 1.0, jnp.float32)
    rows = NP // NS
    pltpu.sync_copy(zvec_hbm.at[pl.ds(sid * rows, rows)],
                    deg_sh.at[pl.ds(sid * rows, rows)])
    plsc.subcore_barrier()

    rbase = wid * ECPW

    def outer(ob, carry):
        row0 = rbase + ob * IDX_BLK
        pltpu.sync_copy(dst2d_hbm.at[pl.ds(row0, IDX_BLK)], idx_v)
        for j in range(IDX_BLK):
            pltpu.sync_copy(ones_v, deg_sh.at[idx_v.at[j]], add=True)
        return carry

    lax.fori_loop(0, ECPW // IDX_BLK, outer, 0)
    plsc.subcore_barrier()
    pltpu.sync_copy(deg_sh.at[pl.ds(sid * rows, rows)],
                    deg_hbm.at[pl.ds(cid * NP + sid * rows, rows)])


NBUF = 4
IDXB = 16


@functools.partial(
    pl.kernel,
    out_type=jax.ShapeDtypeStruct((NC * NP, D), jnp.float32),
    mesh=_mesh,
    scratch_types=[
        pltpu.VMEM((IDXB, EC), jnp.int32),
        pltpu.VMEM((IDXB, EC), jnp.int32),
        pltpu.VMEM((EC, D), jnp.float32),
        pltpu.VMEM((EC, D), jnp.float32),
        pltpu.VMEM((EC, D), jnp.float32),
        pltpu.VMEM((EC, D), jnp.float32),
        pltpu.VMEM_SHARED((NP, D), jnp.float32),
        pltpu.SemaphoreType.DMA,
        pltpu.SemaphoreType.DMA,
        pltpu.SemaphoreType.DMA,
        pltpu.SemaphoreType.DMA,
        pltpu.SemaphoreType.DMA,
        pltpu.SemaphoreType.DMA,
        pltpu.SemaphoreType.DMA,
        pltpu.SemaphoreType.DMA,
    ],
)
def _scatter_kernel(g_hbm, src2d_hbm, dst2d_hbm, zrows_hbm, acc_hbm,
                    sidx, didx, r0, r1, r2, r3, acc_sh,
                    gs0, gs1, gs2, gs3, ss0, ss1, ss2, ss3):
    cid = lax.axis_index("c")
    sid = lax.axis_index("s")
    wid = sid * NC + cid
    rbase = wid * ECPW
    rows_b = (r0, r1, r2, r3)
    gsem = (gs0, gs1, gs2, gs3)
    ssem = (ss0, ss1, ss2, ss3)

    for r in range(NP // CHUNK // NS):
        row0 = (sid * (NP // CHUNK // NS) + r) * CHUNK
        pltpu.sync_copy(zrows_hbm, acc_sh.at[pl.ds(row0, CHUNK)])
    plsc.subcore_barrier()

    def g_start(k, b):
        pltpu.async_copy(g_hbm.at[sidx.at[k]], rows_b[b], gsem[b])

    def g_wait(b):
        pltpu.make_async_copy(g_hbm.at[sidx.at[0]], rows_b[b], gsem[b]).wait()

    def s_start(k, b):
        pltpu.async_copy(rows_b[b], acc_sh.at[didx.at[k]], ssem[b], add=True)

    def s_wait(b):
        pltpu.make_async_copy(rows_b[b], acc_sh.at[didx.at[0]],
                              ssem[b]).wait()

    def block(ob, carry):
        row0 = rbase + ob * IDXB
        pltpu.sync_copy(src2d_hbm.at[pl.ds(row0, IDXB)], sidx)
        pltpu.sync_copy(dst2d_hbm.at[pl.ds(row0, IDXB)], didx)
        g_start(0, 0)
        g_start(1, 1)
        for k in range(IDXB):
            kn = k + 2
            if kn < IDXB:
                if kn >= NBUF:
                    s_wait(kn % NBUF)
                g_start(kn, kn % NBUF)
            g_wait(k % NBUF)
            s_start(k, k % NBUF)
        for b in range(NBUF):
            s_wait((IDXB - NBUF + b) % NBUF)
        return carry

    lax.fori_loop(0, ECPW // IDXB, block, 0)
    plsc.subcore_barrier()
    for r in range(NP // CHUNK // NS):
        row0 = (sid * (NP // CHUNK // NS) + r) * CHUNK
        pltpu.sync_copy(acc_sh.at[pl.ds(row0, CHUNK)],
                        acc_hbm.at[pl.ds(cid * NP + row0, CHUNK)])


IPSB = 4
SPSB = IPSB * S
NSB = (B // NW) // IPSB


@functools.partial(
    pl.kernel,
    out_type=jax.ShapeDtypeStruct((B * S,), jnp.float32),
    mesh=_mesh,
    scratch_types=[
        pltpu.VMEM((B // NW,), jnp.int32),
        pltpu.VMEM((B // NW * S,), jnp.int32),
        pltpu.VMEM((B // NW, D), jnp.float32),
        pltpu.VMEM((SPSB, D), jnp.float32),
        pltpu.VMEM((SPSB, D), jnp.float32),
        pltpu.VMEM((16, 16), jnp.float32),
        pltpu.VMEM((B // NW * S,), jnp.float32),
        pltpu.SemaphoreType.DMA,
        pltpu.SemaphoreType.DMA,
        pltpu.SemaphoreType.DMA,
    ],
)
def _score_kernel(upd_hbm, items_hbm, samps_hbm, out_hbm,
                  it_idx, sp_idx, itbuf, smp0, smp1, stage, score,
                  isem, gs0, gs1):
    cid = lax.axis_index("c")
    sid = lax.axis_index("s")
    wid = sid * NC + cid
    ipw = B // NW
    spw = ipw * S
    smp_b = (smp0, smp1)
    gsem = (gs0, gs1)

    pltpu.sync_copy(items_hbm.at[pl.ds(wid * ipw, ipw)], it_idx)
    pltpu.sync_copy(samps_hbm.at[pl.ds(wid * spw, spw)], sp_idx)
    pltpu.async_copy(upd_hbm.at[it_idx], itbuf, isem).wait()

    rows16 = lax.iota(jnp.int32, 16)
    perms = [rows16 ^ k for k in (8, 4, 2, 1)]

    def g_start(sb, b):
        pltpu.async_copy(upd_hbm.at[sp_idx.at[pl.ds(sb * SPSB, SPSB)]],
                         smp_b[b], gsem[b])

    def g_wait(b):
        pltpu.make_async_copy(upd_hbm.at[sp_idx.at[pl.ds(0, SPSB)]],
                              smp_b[b], gsem[b]).wait()

    def compute(sb, b):
        buf = smp_b[b]
        vacc = jnp.zeros((16,), jnp.float32)
        for i in range(IPSB):
            irow = sb * IPSB + i
            itv = [itbuf[irow, pl.ds(v * 16, 16)] for v in range(8)]
            for s in range(S):
                d = i * S + s
                t = itv[0] * buf[d, pl.ds(0, 16)]
                for v in range(1, 8):
                    t = t + itv[v] * buf[d, pl.ds(v * 16, 16)]
                for p in perms:
                    t = t + t[p]
                vacc = jnp.where(rows16 == (d % 16), t, vacc)
                if d % 16 == 15:
                    score[pl.ds(sb * SPSB + (d // 16) * 16, 16)] = vacc

    g_start(0, 0)

    def body(g, carry):
        sb = 2 * g
        g_start(sb + 1, 1)
        g_wait(0)
        compute(sb, 0)
        g_start(sb + 2, 0)
        g_wait(1)
        compute(sb + 1, 1)
        return carry

    lax.fori_loop(0, NSB // 2 - 1, body, 0)
    g_start(NSB - 1, 1)
    g_wait(0)
    compute(NSB - 2, 0)
    g_wait(1)
    compute(NSB - 1, 1)

    pltpu.sync_copy(score, out_hbm.at[pl.ds(wid * spw, spw)])


_RB = 2048


def _tc1_body(emb_ref, w_ref, deg_ref, g_ref, dinv_ref):
    deg = deg_ref[0] + deg_ref[1] + 1.0
    dinv = lax.rsqrt(deg)
    dinv_ref[...] = dinv
    h = jnp.dot(emb_ref[...], w_ref[...], preferred_element_type=jnp.float32)
    g_ref[...] = h * dinv


_tc1 = pl.pallas_call(
    _tc1_body,
    grid=(NP // _RB,),
    in_specs=[
        pl.BlockSpec((_RB, D), lambda i: (i, 0)),
        pl.BlockSpec((D, D), lambda i: (0, 0)),
        pl.BlockSpec((NC, _RB, 1), lambda i: (0, i, 0)),
    ],
    out_specs=[
        pl.BlockSpec((_RB, D), lambda i: (i, 0)),
        pl.BlockSpec((_RB, 1), lambda i: (i, 0)),
    ],
    out_shape=[
        jax.ShapeDtypeStruct((NP, D), jnp.float32),
        jax.ShapeDtypeStruct((NP, 1), jnp.float32),
    ],
)


def _tc2_body(acc_ref, g_ref, dinv_ref, b_ref, upd_ref):
    upd_ref[...] = (dinv_ref[...] * (acc_ref[0] + acc_ref[1] + g_ref[...])
                    + b_ref[...])


_tc2 = pl.pallas_call(
    _tc2_body,
    grid=(NP // _RB,),
    in_specs=[
        pl.BlockSpec((NC, _RB, D), lambda i: (0, i, 0)),
        pl.BlockSpec((_RB, D), lambda i: (i, 0)),
        pl.BlockSpec((_RB, 1), lambda i: (i, 0)),
        pl.BlockSpec((1, D), lambda i: (0, 0)),
    ],
    out_specs=pl.BlockSpec((_RB, D), lambda i: (i, 0)),
    out_shape=jax.ShapeDtypeStruct((NP, D), jnp.float32),
)

def kernel(items, samples, edge_index, emb_weight, W, b):
    f32 = jnp.float32
    i32 = jnp.int32
    src = edge_index[0].astype(i32)
    dst = edge_index[1].astype(i32)
    npad = EP - E
    pad = (jnp.arange(npad, dtype=i32) % (NP - N_NODES)) + N_NODES
    src2d = jnp.concatenate([src, pad]).reshape(EP // EC, EC)
    dst2d = jnp.concatenate([dst, pad]).reshape(EP // EC, EC)
    emb_p = jnp.pad(emb_weight.astype(f32), ((0, NP - N_NODES), (0, 0)))

    zvec = jnp.zeros((NP,), f32)
    zrows = jnp.zeros((CHUNK, D), f32)

    deg = _deg_kernel(dst2d, zvec)
    g, dinv = _tc1(emb_p, W.astype(f32), deg.reshape(NC, NP, 1))
    acc = _scatter_kernel(g, src2d, dst2d, zrows)
    upd = _tc2(acc.reshape(NC, NP, D), g, dinv, b.astype(f32).reshape(1, D))

    flat = _score_kernel(upd, items.astype(i32),
                         samples.astype(i32).reshape(-1))
    return flat.reshape(B, S)


# speedup vs baseline: 1.2122x; 1.0357x over previous
import functools

import jax
import jax.numpy as jnp
from jax import lax
from jax.experimental import pallas as pl
from jax.experimental.pallas import tpu as pltpu
from jax.experimental.pallas import tpu_sc as plsc

N_NODES = 10000
D = 128
NP = 10240
E = 320000
B = 4096
S = 20
NC, NS = 2, 16
NW = NC * NS
CHUNK = 128
EC = 80
ECPW = 128
EP = NW * ECPW * EC
IDX_BLK = 8
G_TOT = B * (S + 1)
GPW_CH = 24
G_PAD = NW * GPW_CH * CHUNK

_mesh = plsc.VectorSubcoreMesh(core_axis_name="c", subcore_axis_name="s")


@functools.partial(
    pl.kernel,
    out_type=jax.ShapeDtypeStruct((NC * NP,), jnp.float32),
    mesh=_mesh,
    scratch_types=[
        pltpu.VMEM((IDX_BLK, EC), jnp.int32),
        pltpu.VMEM((EC,), jnp.float32),
        pltpu.VMEM_SHARED((NP,), jnp.float32),
    ],
)
def _deg_kernel(dst2d_hbm, zvec_hbm, deg_hbm, idx_v, ones_v, deg_sh):
    cid = lax.axis_index("c")
    sid = lax.axis_index("s")
    wid = sid * NC + cid
    for j in range(EC // 16):
        ones_v[pl.ds(j * 16, 16)] = jnp.full((16,), 1.0, jnp.float32)
    rows = NP // NS
    pltpu.sync_copy(zvec_hbm.at[pl.ds(sid * rows, rows)],
                    deg_sh.at[pl.ds(sid * rows, rows)])
    plsc.subcore_barrier()

    rbase = wid * ECPW

    def outer(ob, carry):
        row0 = rbase + ob * IDX_BLK
        pltpu.sync_copy(dst2d_hbm.at[pl.ds(row0, IDX_BLK)], idx_v)
        for j in range(IDX_BLK):
            pltpu.sync_copy(ones_v, deg_sh.at[idx_v.at[j]], add=True)
        return carry

    lax.fori_loop(0, ECPW // IDX_BLK, outer, 0)
    plsc.subcore_barrier()
    pltpu.sync_copy(deg_sh.at[pl.ds(sid * rows, rows)],
                    deg_hbm.at[pl.ds(cid * NP + sid * rows, rows)])


NBUF = 4
IDXB = 32


@functools.partial(
    pl.kernel,
    out_type=jax.ShapeDtypeStruct((NC * NP, D), jnp.float32),
    mesh=_mesh,
    scratch_types=[
        pltpu.VMEM((IDXB, EC), jnp.int32),
        pltpu.VMEM((IDXB, EC), jnp.int32),
        pltpu.VMEM((EC, D), jnp.float32),
        pltpu.VMEM((EC, D), jnp.float32),
        pltpu.VMEM((EC, D), jnp.float32),
        pltpu.VMEM((EC, D), jnp.float32),
        pltpu.VMEM_SHARED((NP, D), jnp.float32),
        pltpu.SemaphoreType.DMA,
        pltpu.SemaphoreType.DMA,
        pltpu.SemaphoreType.DMA,
        pltpu.SemaphoreType.DMA,
        pltpu.SemaphoreType.DMA,
        pltpu.SemaphoreType.DMA,
        pltpu.SemaphoreType.DMA,
        pltpu.SemaphoreType.DMA,
    ],
)
def _scatter_kernel(g_hbm, src2d_hbm, dst2d_hbm, zrows_hbm, acc_hbm,
                    sidx, didx, r0, r1, r2, r3, acc_sh,
                    gs0, gs1, gs2, gs3, ss0, ss1, ss2, ss3):
    cid = lax.axis_index("c")
    sid = lax.axis_index("s")
    wid = sid * NC + cid
    rbase = wid * ECPW
    rows_b = (r0, r1, r2, r3)
    gsem = (gs0, gs1, gs2, gs3)
    ssem = (ss0, ss1, ss2, ss3)

    for r in range(NP // CHUNK // NS):
        row0 = (sid * (NP // CHUNK // NS) + r) * CHUNK
        pltpu.sync_copy(zrows_hbm, acc_sh.at[pl.ds(row0, CHUNK)])
    plsc.subcore_barrier()

    def g_start(k, b):
        pltpu.async_copy(g_hbm.at[sidx.at[k]], rows_b[b], gsem[b])

    def g_wait(b):
        pltpu.make_async_copy(g_hbm.at[sidx.at[0]], rows_b[b], gsem[b]).wait()

    def s_start(k, b):
        pltpu.async_copy(rows_b[b], acc_sh.at[didx.at[k]], ssem[b], add=True)

    def s_wait(b):
        pltpu.make_async_copy(rows_b[b], acc_sh.at[didx.at[0]],
                              ssem[b]).wait()

    def block(ob, carry):
        row0 = rbase + ob * IDXB
        pltpu.sync_copy(src2d_hbm.at[pl.ds(row0, IDXB)], sidx)
        pltpu.sync_copy(dst2d_hbm.at[pl.ds(row0, IDXB)], didx)
        g_start(0, 0)
        g_start(1, 1)
        for k in range(IDXB):
            kn = k + 2
            if kn < IDXB:
                if kn >= NBUF:
                    s_wait(kn % NBUF)
                g_start(kn, kn % NBUF)
            g_wait(k % NBUF)
            s_start(k, k % NBUF)
        for b in range(NBUF):
            s_wait((IDXB - NBUF + b) % NBUF)
        return carry

    lax.fori_loop(0, ECPW // IDXB, block, 0)
    plsc.subcore_barrier()
    for r in range(NP // CHUNK // NS):
        row0 = (sid * (NP // CHUNK // NS) + r) * CHUNK
        pltpu.sync_copy(acc_sh.at[pl.ds(row0, CHUNK)],
                        acc_hbm.at[pl.ds(cid * NP + row0, CHUNK)])


IPSB = 4
SPSB = IPSB * S
NSB = (B // NW) // IPSB


@functools.partial(
    pl.kernel,
    out_type=jax.ShapeDtypeStruct((B * S,), jnp.float32),
    mesh=_mesh,
    scratch_types=[
        pltpu.VMEM((B // NW,), jnp.int32),
        pltpu.VMEM((B // NW * S,), jnp.int32),
        pltpu.VMEM((B // NW, D), jnp.float32),
        pltpu.VMEM((SPSB, D), jnp.float32),
        pltpu.VMEM((SPSB, D), jnp.float32),
        pltpu.VMEM((16, 16), jnp.float32),
        pltpu.VMEM((B // NW * S,), jnp.float32),
        pltpu.SemaphoreType.DMA,
        pltpu.SemaphoreType.DMA,
        pltpu.SemaphoreType.DMA,
    ],
)
def _score_kernel(upd_hbm, items_hbm, samps_hbm, out_hbm,
                  it_idx, sp_idx, itbuf, smp0, smp1, stage, score,
                  isem, gs0, gs1):
    cid = lax.axis_index("c")
    sid = lax.axis_index("s")
    wid = sid * NC + cid
    ipw = B // NW
    spw = ipw * S
    smp_b = (smp0, smp1)
    gsem = (gs0, gs1)

    pltpu.sync_copy(items_hbm.at[pl.ds(wid * ipw, ipw)], it_idx)
    pltpu.sync_copy(samps_hbm.at[pl.ds(wid * spw, spw)], sp_idx)
    pltpu.async_copy(upd_hbm.at[it_idx], itbuf, isem).wait()

    rows16 = lax.iota(jnp.int32, 16)
    perms = [rows16 ^ k for k in (8, 4, 2, 1)]

    def g_start(sb, b):
        pltpu.async_copy(upd_hbm.at[sp_idx.at[pl.ds(sb * SPSB, SPSB)]],
                         smp_b[b], gsem[b])

    def g_wait(b):
        pltpu.make_async_copy(upd_hbm.at[sp_idx.at[pl.ds(0, SPSB)]],
                              smp_b[b], gsem[b]).wait()

    def compute(sb, b):
        buf = smp_b[b]
        vacc = jnp.zeros((16,), jnp.float32)
        for i in range(IPSB):
            irow = sb * IPSB + i
            itv = [itbuf[irow, pl.ds(v * 16, 16)] for v in range(8)]
            for s in range(S):
                d = i * S + s
                t = itv[0] * buf[d, pl.ds(0, 16)]
                for v in range(1, 8):
                    t = t + itv[v] * buf[d, pl.ds(v * 16, 16)]
                for p in perms:
                    t = t + t[p]
                vacc = jnp.where(rows16 == (d % 16), t, vacc)
                if d % 16 == 15:
                    score[pl.ds(sb * SPSB + (d // 16) * 16, 16)] = vacc

    g_start(0, 0)

    def body(g, carry):
        sb = 2 * g
        g_start(sb + 1, 1)
        g_wait(0)
        compute(sb, 0)
        g_start(sb + 2, 0)
        g_wait(1)
        compute(sb + 1, 1)
        return carry

    lax.fori_loop(0, NSB // 2 - 1, body, 0)
    g_start(NSB - 1, 1)
    g_wait(0)
    compute(NSB - 2, 0)
    g_wait(1)
    compute(NSB - 1, 1)

    pltpu.sync_copy(score, out_hbm.at[pl.ds(wid * spw, spw)])


_RB = 2048


def _tc1_body(emb_ref, w_ref, deg_ref, g_ref, dinv_ref):
    deg = deg_ref[0] + deg_ref[1] + 1.0
    dinv = lax.rsqrt(deg)
    dinv_ref[...] = dinv
    h = jnp.dot(emb_ref[...], w_ref[...], preferred_element_type=jnp.float32)
    g_ref[...] = h * dinv


_tc1 = pl.pallas_call(
    _tc1_body,
    grid=(NP // _RB,),
    in_specs=[
        pl.BlockSpec((_RB, D), lambda i: (i, 0)),
        pl.BlockSpec((D, D), lambda i: (0, 0)),
        pl.BlockSpec((NC, _RB, 1), lambda i: (0, i, 0)),
    ],
    out_specs=[
        pl.BlockSpec((_RB, D), lambda i: (i, 0)),
        pl.BlockSpec((_RB, 1), lambda i: (i, 0)),
    ],
    out_shape=[
        jax.ShapeDtypeStruct((NP, D), jnp.float32),
        jax.ShapeDtypeStruct((NP, 1), jnp.float32),
    ],
)


def _tc2_body(acc_ref, g_ref, dinv_ref, b_ref, upd_ref):
    upd_ref[...] = (dinv_ref[...] * (acc_ref[0] + acc_ref[1] + g_ref[...])
                    + b_ref[...])


_tc2 = pl.pallas_call(
    _tc2_body,
    grid=(NP // _RB,),
    in_specs=[
        pl.BlockSpec((NC, _RB, D), lambda i: (0, i, 0)),
        pl.BlockSpec((_RB, D), lambda i: (i, 0)),
        pl.BlockSpec((_RB, 1), lambda i: (i, 0)),
        pl.BlockSpec((1, D), lambda i: (0, 0)),
    ],
    out_specs=pl.BlockSpec((_RB, D), lambda i: (i, 0)),
    out_shape=jax.ShapeDtypeStruct((NP, D), jnp.float32),
)

def kernel(items, samples, edge_index, emb_weight, W, b):
    f32 = jnp.float32
    i32 = jnp.int32
    src = edge_index[0].astype(i32)
    dst = edge_index[1].astype(i32)
    npad = EP - E
    pad = (jnp.arange(npad, dtype=i32) % (NP - N_NODES)) + N_NODES
    src2d = jnp.concatenate([src, pad]).reshape(EP // EC, EC)
    dst2d = jnp.concatenate([dst, pad]).reshape(EP // EC, EC)
    emb_p = jnp.pad(emb_weight.astype(f32), ((0, NP - N_NODES), (0, 0)))

    zvec = jnp.zeros((NP,), f32)
    zrows = jnp.zeros((CHUNK, D), f32)

    deg = _deg_kernel(dst2d, zvec)
    g, dinv = _tc1(emb_p, W.astype(f32), deg.reshape(NC, NP, 1))
    acc = _scatter_kernel(g, src2d, dst2d, zrows)
    upd = _tc2(acc.reshape(NC, NP, D), g, dinv, b.astype(f32).reshape(1, D))

    flat = _score_kernel(upd, items.astype(i32),
                         samples.astype(i32).reshape(-1))
    return flat.reshape(B, S)
